# baseline (speedup 1.0000x reference)
import functools

import jax
import jax.numpy as jnp
from jax import lax
from jax.experimental import pallas as pl

_W0 = 30.0
_LANES = 128


def _feat_body(grid_ref, w1_ref, b1_ref, w2_ref, b2_ref, wout_ref, bout_ref,
               featT_ref, outT_ref):
    t1 = jnp.dot(grid_ref[...], w1_ref[...],
                 preferred_element_type=jnp.float32) + b1_ref[...]
    hdn = jnp.sin(_W0 * t1)
    t2 = jnp.dot(hdn, w2_ref[...],
                 preferred_element_type=jnp.float32) + b2_ref[...]
    ft = jnp.sin(_W0 * t2)
    ot = jnp.dot(ft, wout_ref[...],
                 preferred_element_type=jnp.float32) + bout_ref[...]
    featT_ref[...] = jnp.transpose(ft, (1, 0))
    outT_ref[...] = jnp.transpose(ot, (1, 0))


def _sort_body_t(n, w, outl, v_ref, vals_ref, idx_ref, cx_ref, cy_ref):
    v = v_ref[...]
    b, rows, lanes = v.shape
    row_io = lax.broadcasted_iota(jnp.int32, v.shape, 1)
    lane_io = lax.broadcasted_iota(jnp.int32, v.shape, 2)
    pos = lane_io * rows + row_io
    ix = row_io * lanes + lane_io

    kk = 2
    while kk <= n:
        jj = kk // 2
        while jj >= 1:
            if jj >= rows:
                axis, amt = 2, jj // rows
            else:
                axis, amt = 1, jj
            isb = (pos & jj) != 0
            pv = jnp.where(isb, jnp.roll(v, amt, axis), jnp.roll(v, -amt, axis))
            pix = jnp.where(isb, jnp.roll(ix, amt, axis), jnp.roll(ix, -amt, axis))
            lt_sp = (v > pv) | ((v == pv) & (ix < pix))
            keep = lt_sp == (((pos & kk) == 0) ^ isb)
            v = jnp.where(keep, v, pv)
            ix = jnp.where(keep, ix, pix)
            jj //= 2
        kk *= 2

    top_v = v[:, :, :outl]
    top_i = ix[:, :, :outl]
    vals_ref[...] = top_v
    idx_ref[...] = top_i
    gr = top_i // w
    gc = top_i - gr * w
    step = jnp.float32(2.0 / (w - 1))
    cx_ref[...] = gr.astype(jnp.float32) * step - 1.0
    cy_ref[...] = gc.astype(jnp.float32) * step - 1.0


def kernel(inputs, W1, b1, W2, b2, Wout, bout):
    b, c, h, w = inputs.shape
    hw = h * w
    xs = jnp.linspace(-1.0, 1.0, h)
    ys = jnp.linspace(-1.0, 1.0, w)
    gx, gy = jnp.meshgrid(xs, ys, indexing='ij')
    grid = jnp.stack([gx, gy], axis=-1).reshape(h * w, 2)
    f = W2.shape[1]
    ch = hw // 8
    featT, outT = pl.pallas_call(
        _feat_body,
        grid=(hw // ch,),
        in_specs=[
            pl.BlockSpec((ch, 2), lambda i: (i, 0)),
            pl.BlockSpec((2, f), lambda i: (0, 0)),
            pl.BlockSpec((1, f), lambda i: (0, 0)),
            pl.BlockSpec((f, f), lambda i: (0, 0)),
            pl.BlockSpec((1, f), lambda i: (0, 0)),
            pl.BlockSpec((f, c), lambda i: (0, 0)),
            pl.BlockSpec((1, c), lambda i: (0, 0)),
        ],
        out_specs=[
            pl.BlockSpec((f, ch), lambda i: (0, i)),
            pl.BlockSpec((c, ch), lambda i: (0, i)),
        ],
        out_shape=[
            jax.ShapeDtypeStruct((f, hw), jnp.float32),
            jax.ShapeDtypeStruct((c, hw), jnp.float32),
        ],
    )(grid, W1, b1[None, :], W2, b2[None, :], Wout, bout[None, :])
    out_t = jnp.broadcast_to(outT[None], (b, c, hw))
    feat_t = jnp.broadcast_to(featT[None], (b, f, hw))
    inp_flat = inputs.reshape(b, c, hw)
    error = inp_flat - out_t
    gradient = -1.0 * feat_t[:, None, :, :] * error[:, :, None, :]
    gradient_bias = -1.0 * error[:, :, None, :]
    gradient = jnp.concatenate([gradient, gradient_bias], axis=2)
    gradient = gradient.reshape(b, -1, hw)
    gradient_norm = jnp.linalg.norm(gradient, axis=1)

    k = int(hw * 0.25)
    n = 1 << (hw - 1).bit_length()
    rows = n // _LANES
    outl = -(-(-(-k // rows)) // 8) * 8
    pad = jnp.full((b, n - hw), -1.0, dtype=jnp.float32)
    padded = jnp.concatenate([gradient_norm, pad], axis=1).reshape(b, rows, _LANES)

    body = functools.partial(_sort_body_t, n, w, outl)
    vals, idx, cx, cy = pl.pallas_call(
        body,
        out_shape=[
            jax.ShapeDtypeStruct((b, rows, outl), jnp.float32),
            jax.ShapeDtypeStruct((b, rows, outl), jnp.int32),
            jax.ShapeDtypeStruct((b, rows, outl), jnp.float32),
            jax.ShapeDtypeStruct((b, rows, outl), jnp.float32),
        ],
    )(padded)

    unt = lambda a: jnp.transpose(a, (0, 2, 1)).reshape(b, rows * outl)[:, :k]
    topk_vals = unt(vals)
    gradncp_index = unt(idx)
    gradncp_coord = jnp.stack([unt(cx), unt(cy)], axis=-1)
    return gradncp_coord, topk_vals, gradncp_index


# speedup vs baseline: 1.0023x; 1.0023x over previous
import functools

import jax
import jax.numpy as jnp
from jax import lax
from jax.experimental import pallas as pl

_W0 = 30.0
_LANES = 128


def _feat_body(grid_ref, w1_ref, b1_ref, w2_ref, b2_ref, feat_ref, featT_ref):
    t1 = jnp.dot(grid_ref[...], w1_ref[...],
                 preferred_element_type=jnp.float32) + b1_ref[...]
    hdn = jnp.sin(_W0 * t1)
    t2 = jnp.dot(hdn, w2_ref[...],
                 preferred_element_type=jnp.float32) + b2_ref[...]
    ft = jnp.sin(_W0 * t2)
    feat_ref[...] = ft
    featT_ref[...] = jnp.transpose(ft, (1, 0))


def _sort_body_t(n, w, outl, v_ref, vals_ref, idx_ref, cx_ref, cy_ref):
    v = v_ref[...]
    b, rows, lanes = v.shape
    row_io = lax.broadcasted_iota(jnp.int32, v.shape, 1)
    lane_io = lax.broadcasted_iota(jnp.int32, v.shape, 2)
    pos = lane_io * rows + row_io
    ix = row_io * lanes + lane_io

    kk = 2
    while kk <= n:
        jj = kk // 2
        while jj >= 1:
            if jj >= rows:
                axis, amt = 2, jj // rows
            else:
                axis, amt = 1, jj
            isb = (pos & jj) != 0
            pv = jnp.where(isb, jnp.roll(v, amt, axis), jnp.roll(v, -amt, axis))
            pix = jnp.where(isb, jnp.roll(ix, amt, axis), jnp.roll(ix, -amt, axis))
            lt_sp = (v > pv) | ((v == pv) & (ix < pix))
            keep = lt_sp == (((pos & kk) == 0) ^ isb)
            v = jnp.where(keep, v, pv)
            ix = jnp.where(keep, ix, pix)
            jj //= 2
        kk *= 2

    top_v = v[:, :, :outl]
    top_i = ix[:, :, :outl]
    vals_ref[...] = top_v
    idx_ref[...] = top_i
    gr = top_i // w
    gc = top_i - gr * w
    step = jnp.float32(2.0 / (w - 1))
    cx_ref[...] = gr.astype(jnp.float32) * step - 1.0
    cy_ref[...] = gc.astype(jnp.float32) * step - 1.0


def kernel(inputs, W1, b1, W2, b2, Wout, bout):
    b, c, h, w = inputs.shape
    hw = h * w
    xs = jnp.linspace(-1.0, 1.0, h)
    ys = jnp.linspace(-1.0, 1.0, w)
    gx, gy = jnp.meshgrid(xs, ys, indexing='ij')
    grid = jnp.stack([gx, gy], axis=-1).reshape(h * w, 2)
    f = W2.shape[1]
    ch = hw // 8
    feat1, featT = pl.pallas_call(
        _feat_body,
        grid=(hw // ch,),
        in_specs=[
            pl.BlockSpec((ch, 2), lambda i: (i, 0)),
            pl.BlockSpec((2, f), lambda i: (0, 0)),
            pl.BlockSpec((1, f), lambda i: (0, 0)),
            pl.BlockSpec((f, f), lambda i: (0, 0)),
            pl.BlockSpec((1, f), lambda i: (0, 0)),
        ],
        out_specs=[
            pl.BlockSpec((ch, f), lambda i: (i, 0)),
            pl.BlockSpec((f, ch), lambda i: (0, i)),
        ],
        out_shape=[
            jax.ShapeDtypeStruct((hw, f), jnp.float32),
            jax.ShapeDtypeStruct((f, hw), jnp.float32),
        ],
    )(grid, W1, b1[None, :], W2, b2[None, :])
    out1 = feat1 @ Wout + bout
    out_t = jnp.broadcast_to(jnp.transpose(out1, (1, 0))[None], (b, c, hw))
    feat_t = jnp.broadcast_to(featT[None], (b, f, hw))
    inp_flat = inputs.reshape(b, c, hw)
    error = inp_flat - out_t
    gradient = -1.0 * feat_t[:, None, :, :] * error[:, :, None, :]
    gradient_bias = -1.0 * error[:, :, None, :]
    gradient = jnp.concatenate([gradient, gradient_bias], axis=2)
    gradient = gradient.reshape(b, -1, hw)
    gradient_norm = jnp.linalg.norm(gradient, axis=1)

    k = int(hw * 0.25)
    n = 1 << (hw - 1).bit_length()
    rows = n // _LANES
    outl = -(-(-(-k // rows)) // 8) * 8
    pad = jnp.full((b, n - hw), -1.0, dtype=jnp.float32)
    padded = jnp.concatenate([gradient_norm, pad], axis=1).reshape(b, rows, _LANES)

    body = functools.partial(_sort_body_t, n, w, outl)
    vals, idx, cx, cy = pl.pallas_call(
        body,
        out_shape=[
            jax.ShapeDtypeStruct((b, rows, outl), jnp.float32),
            jax.ShapeDtypeStruct((b, rows, outl), jnp.int32),
            jax.ShapeDtypeStruct((b, rows, outl), jnp.float32),
            jax.ShapeDtypeStruct((b, rows, outl), jnp.float32),
        ],
    )(padded)

    unt = lambda a: jnp.transpose(a, (0, 2, 1)).reshape(b, rows * outl)[:, :k]
    topk_vals = unt(vals)
    gradncp_index = unt(idx)
    gradncp_coord = jnp.stack([unt(cx), unt(cy)], axis=-1)
    return gradncp_coord, topk_vals, gradncp_index
